# Initial kernel scaffold; baseline (speedup 1.0000x reference)
import functools

import jax
import jax.numpy as jnp
from jax import lax
from jax.experimental import pallas as pl
from jax.experimental.pallas import tpu as pltpu
from jax.experimental.pallas import tpu_sc as plsc

N = 10000
E = 320000
D_IN = 128
HID = 128

NC = 2
NS = 16
CHUNK = 128

N_PAD = 10240
ROWS_PER_TILE = N_PAD // NS
ROW_CHUNKS = ROWS_PER_TILE // CHUNK

E_PAD = 323584
EPT_HIST = E_PAD // (NC * NS)
HIST_ITERS = EPT_HIST // CHUNK
EPT_SCAT = E_PAD // NS
SCAT_ITERS = EPT_SCAT // CHUNK

_mesh = plsc.VectorSubcoreMesh(core_axis_name="c", subcore_axis_name="s")


def _fill(ref, n, value):
    v = jnp.full((16,), value, dtype=ref.dtype)
    for k in range(n // 16):
        ref[pl.ds(k * 16, 16)] = v


@functools.partial(
    pl.kernel,
    out_type=(
        jax.ShapeDtypeStruct((N_PAD,), jnp.float32),
        jax.ShapeDtypeStruct((N_PAD,), jnp.float32),
        jax.ShapeDtypeStruct((N_PAD, D_IN), jnp.float32),
    ),
    mesh=_mesh,
    scratch_types=(
        pltpu.VMEM_SHARED((N_PAD,), jnp.float32),
        pltpu.VMEM((CHUNK,), jnp.int32),
        pltpu.VMEM((CHUNK,), jnp.int32),
        pltpu.VMEM((CHUNK, D_IN), jnp.float32),
        pltpu.VMEM((ROWS_PER_TILE,), jnp.float32),
        pltpu.VMEM((CHUNK,), jnp.float32),
        pltpu.SemaphoreType.DMA,
    ),
)
def _prep_kernel(dst_hbm, x_hbm, emb_hbm, deg0_hbm, deg1_hbm, h0_hbm,
                 deg_sh, didx, xidx, rows, zbuf, ones, sem):
    c = lax.axis_index("c")
    s = lax.axis_index("s")
    wid = c * NS + s

    _fill(zbuf, ROWS_PER_TILE, 0.0)
    _fill(ones, CHUNK, 1.0)
    pltpu.sync_copy(zbuf, deg_sh.at[pl.ds(s * ROWS_PER_TILE, ROWS_PER_TILE)])
    plsc.subcore_barrier()

    def hist_step(i, carry):
        base = wid * EPT_HIST + i * CHUNK
        pltpu.sync_copy(dst_hbm.at[pl.ds(base, CHUNK)], didx)
        pltpu.sync_copy(ones, deg_sh.at[didx], add=True)
        return carry

    lax.fori_loop(0, HIST_ITERS, hist_step, 0)

    @pl.when(c == 0)
    def _():
        def gather_step(j, carry):
            base = s * ROWS_PER_TILE + j * CHUNK
            pltpu.sync_copy(x_hbm.at[pl.ds(base, CHUNK)], xidx)
            pltpu.async_copy(emb_hbm.at[xidx], rows, sem).wait()
            pltpu.sync_copy(rows, h0_hbm.at[pl.ds(base, CHUNK)])
            return carry

        lax.fori_loop(0, ROW_CHUNKS, gather_step, 0)

    plsc.subcore_barrier()

    sl = pl.ds(s * ROWS_PER_TILE, ROWS_PER_TILE)
    pltpu.sync_copy(deg_sh.at[sl], zbuf)

    @pl.when(c == 0)
    def _():
        pltpu.sync_copy(zbuf, deg0_hbm.at[sl])

    @pl.when(c == 1)
    def _():
        pltpu.sync_copy(zbuf, deg1_hbm.at[sl])


def _make_scatter(dh):
    @functools.partial(
        pl.kernel,
        out_type=(
            jax.ShapeDtypeStruct((N_PAD, dh), jnp.float32),
            jax.ShapeDtypeStruct((N_PAD, dh), jnp.float32),
        ),
        mesh=_mesh,
        scratch_types=(
            pltpu.VMEM_SHARED((N_PAD, dh), jnp.float32),
            pltpu.VMEM((CHUNK,), jnp.int32),
            pltpu.VMEM((CHUNK,), jnp.int32),
            pltpu.VMEM((CHUNK, dh), jnp.float32),
            pltpu.SemaphoreType.DMA,
        ),
    )
    def scatter(src_hbm, dst_hbm, ga_hbm, gb_hbm, sa_hbm, sb_hbm,
                acc, sidx, didx, rows, sem):
        c = lax.axis_index("c")
        s = lax.axis_index("s")

        def run(g_hbm, out_hbm):
            for j in range(ROW_CHUNKS):
                sl = pl.ds(s * ROWS_PER_TILE + j * CHUNK, CHUNK)
                pltpu.sync_copy(g_hbm.at[sl], rows)
                pltpu.sync_copy(rows, acc.at[sl])
            plsc.subcore_barrier()

            def edge_step(i, carry):
                base = s * EPT_SCAT + i * CHUNK
                pltpu.sync_copy(src_hbm.at[pl.ds(base, CHUNK)], sidx)
                pltpu.sync_copy(dst_hbm.at[pl.ds(base, CHUNK)], didx)
                pltpu.async_copy(g_hbm.at[sidx], rows, sem).wait()
                pltpu.sync_copy(rows, acc.at[didx], add=True)
                return carry

            lax.fori_loop(0, SCAT_ITERS, edge_step, 0)
            plsc.subcore_barrier()

            for j in range(ROW_CHUNKS):
                sl = pl.ds(s * ROWS_PER_TILE + j * CHUNK, CHUNK)
                pltpu.sync_copy(acc.at[sl], rows)
                pltpu.sync_copy(rows, out_hbm.at[sl])

        @pl.when(c == 0)
        def _():
            run(ga_hbm, sa_hbm)

        @pl.when(c == 1)
        def _():
            run(gb_hbm, sb_hbm)

    return scatter


_scatter_h = _make_scatter(2 * HID // 2)
_scatter_q = _make_scatter(HID // 2)


def _tc_layer1(dega, degb, h0, w1):
    def body(dega_ref, degb_ref, h0_ref, w1_ref, dinv_ref, ga_ref, gb_ref):
        deg = dega_ref[:] + degb_ref[:] + 1.0
        dinv = lax.rsqrt(deg)
        row = lax.broadcasted_iota(jnp.int32, (N_PAD, 1), 0)
        dinv = jnp.where(row < N, dinv, 0.0)
        dinv_ref[:] = dinv
        g = jnp.dot(h0_ref[:], w1_ref[:],
                    preferred_element_type=jnp.float32) * dinv
        ga_ref[:] = g[:, :HID]
        gb_ref[:] = g[:, HID:]

    return pl.pallas_call(
        body,
        out_shape=(
            jax.ShapeDtypeStruct((N_PAD, 1), jnp.float32),
            jax.ShapeDtypeStruct((N_PAD, HID), jnp.float32),
            jax.ShapeDtypeStruct((N_PAD, HID), jnp.float32),
        ),
    )(dega, degb, h0, w1)


def _tc_layer2(sa, sb, dinv, b1, w2):
    def body(sa_ref, sb_ref, dinv_ref, b1_ref, w2_ref, ga_ref, gb_ref):
        s1 = jnp.concatenate([sa_ref[:], sb_ref[:]], axis=1)
        h1 = jnp.maximum(dinv_ref[:] * s1 + b1_ref[:], 0.0)
        g = jnp.dot(h1, w2_ref[:],
                    preferred_element_type=jnp.float32) * dinv_ref[:]
        ga_ref[:] = g[:, :HID // 2]
        gb_ref[:] = g[:, HID // 2:]

    return pl.pallas_call(
        body,
        out_shape=(
            jax.ShapeDtypeStruct((N_PAD, HID // 2), jnp.float32),
            jax.ShapeDtypeStruct((N_PAD, HID // 2), jnp.float32),
        ),
    )(sa, sb, dinv, b1, w2)


def _tc_out(sa, sb, dinv, b2):
    def body(sa_ref, sb_ref, dinv_ref, b2_ref, z_ref):
        s2 = jnp.concatenate([sa_ref[:], sb_ref[:]], axis=1)
        z_ref[:] = dinv_ref[:] * s2 + b2_ref[:]

    return pl.pallas_call(
        body,
        out_shape=jax.ShapeDtypeStruct((N_PAD, HID), jnp.float32),
    )(sa, sb, dinv, b2)


@jax.jit
def kernel(x, edge_index, emb, W1, b1, W2, b2):
    src = edge_index[0].astype(jnp.int32)
    dst = edge_index[1].astype(jnp.int32)
    pad = jnp.full((E_PAD - E,), N, dtype=jnp.int32)
    srcp = jnp.concatenate([src, pad])
    dstp = jnp.concatenate([dst, pad])
    xp = jnp.concatenate(
        [x[:, 0].astype(jnp.int32), jnp.zeros((N_PAD - N,), jnp.int32)])

    deg0, deg1, h0 = _prep_kernel(dstp, xp, emb)
    dinv, g1a, g1b = _tc_layer1(deg0.reshape(N_PAD, 1),
                                deg1.reshape(N_PAD, 1), h0, W1)
    s1a, s1b = _scatter_h(srcp, dstp, g1a, g1b)
    g2a, g2b = _tc_layer2(s1a, s1b, dinv, b1.reshape(1, 2 * HID), W2)
    s2a, s2b = _scatter_q(srcp, dstp, g2a, g2b)
    z = _tc_out(s2a, s2b, dinv, b2.reshape(1, HID))
    return z[:N]


# speedup vs baseline: 9.5614x; 9.5614x over previous
import functools

import jax
import jax.numpy as jnp
from jax import lax
from jax.experimental import pallas as pl
from jax.experimental.pallas import tpu as pltpu
from jax.experimental.pallas import tpu_sc as plsc

N = 10000
E = 320000
D_IN = 128
HID = 128

NC = 2
NS = 16
CHUNK = 128

N_PAD = 10240
ROWS_PER_TILE = N_PAD // NS
ROW_CHUNKS = ROWS_PER_TILE // CHUNK

E_PAD = 323584
EPT_HIST = E_PAD // (NC * NS)
HIST_ITERS = EPT_HIST // CHUNK
EPT_SCAT = E_PAD // NS
SCAT_ITERS = EPT_SCAT // CHUNK

_mesh = plsc.VectorSubcoreMesh(core_axis_name="c", subcore_axis_name="s")


def _fill(ref, n, value):
    v = jnp.full((16,), value, dtype=ref.dtype)
    for k in range(n // 16):
        ref[pl.ds(k * 16, 16)] = v


@functools.partial(
    pl.kernel,
    out_type=(
        jax.ShapeDtypeStruct((N_PAD,), jnp.float32),
        jax.ShapeDtypeStruct((N_PAD,), jnp.float32),
        jax.ShapeDtypeStruct((N_PAD, D_IN), jnp.float32),
    ),
    mesh=_mesh,
    scratch_types=(
        pltpu.VMEM_SHARED((N_PAD,), jnp.float32),
        pltpu.VMEM((CHUNK,), jnp.int32),
        pltpu.VMEM((CHUNK,), jnp.int32),
        pltpu.VMEM((CHUNK, D_IN), jnp.float32),
        pltpu.VMEM((ROWS_PER_TILE,), jnp.float32),
        pltpu.VMEM((CHUNK,), jnp.float32),
        pltpu.SemaphoreType.DMA,
    ),
)
def _prep_kernel(dst_hbm, x_hbm, emb_hbm, deg0_hbm, deg1_hbm, h0_hbm,
                 deg_sh, didx, xidx, rows, zbuf, ones, sem):
    c = lax.axis_index("c")
    s = lax.axis_index("s")
    wid = c * NS + s

    _fill(zbuf, ROWS_PER_TILE, 0.0)
    _fill(ones, CHUNK, 1.0)
    pltpu.sync_copy(zbuf, deg_sh.at[pl.ds(s * ROWS_PER_TILE, ROWS_PER_TILE)])
    plsc.subcore_barrier()

    def hist_step(i, carry):
        base = wid * EPT_HIST + i * CHUNK
        pltpu.sync_copy(dst_hbm.at[pl.ds(base, CHUNK)], didx)
        pltpu.sync_copy(ones, deg_sh.at[didx], add=True)
        return carry

    lax.fori_loop(0, HIST_ITERS, hist_step, 0)

    @pl.when(c == 0)
    def _():
        def gather_step(j, carry):
            base = s * ROWS_PER_TILE + j * CHUNK
            pltpu.sync_copy(x_hbm.at[pl.ds(base, CHUNK)], xidx)
            pltpu.async_copy(emb_hbm.at[xidx], rows, sem).wait()
            pltpu.sync_copy(rows, h0_hbm.at[pl.ds(base, CHUNK)])
            return carry

        lax.fori_loop(0, ROW_CHUNKS, gather_step, 0)

    plsc.subcore_barrier()

    sl = pl.ds(s * ROWS_PER_TILE, ROWS_PER_TILE)
    pltpu.sync_copy(deg_sh.at[sl], zbuf)

    @pl.when(c == 0)
    def _():
        pltpu.sync_copy(zbuf, deg0_hbm.at[sl])

    @pl.when(c == 1)
    def _():
        pltpu.sync_copy(zbuf, deg1_hbm.at[sl])


def _make_scatter(dh):
    @functools.partial(
        pl.kernel,
        out_type=(
            jax.ShapeDtypeStruct((N_PAD, dh), jnp.float32),
            jax.ShapeDtypeStruct((N_PAD, dh), jnp.float32),
        ),
        mesh=_mesh,
        scratch_types=(
            pltpu.VMEM_SHARED((N_PAD, dh), jnp.float32),
            pltpu.VMEM((CHUNK,), jnp.int32),
            pltpu.VMEM((CHUNK,), jnp.int32),
            pltpu.VMEM((CHUNK, dh), jnp.float32),
            pltpu.SemaphoreType.DMA,
        ),
    )
    def scatter(src_hbm, dst_hbm, ga_hbm, gb_hbm, sa_hbm, sb_hbm,
                acc, sidx, didx, rows, sem):
        c = lax.axis_index("c")
        s = lax.axis_index("s")

        def run(g_hbm, out_hbm):
            for j in range(ROW_CHUNKS):
                sl = pl.ds(s * ROWS_PER_TILE + j * CHUNK, CHUNK)
                pltpu.sync_copy(g_hbm.at[sl], rows)
                pltpu.sync_copy(rows, acc.at[sl])
            plsc.subcore_barrier()

            def edge_step(i, carry):
                base = s * EPT_SCAT + i * CHUNK
                pltpu.sync_copy(src_hbm.at[pl.ds(base, CHUNK)], sidx)
                pltpu.sync_copy(dst_hbm.at[pl.ds(base, CHUNK)], didx)
                pltpu.async_copy(g_hbm.at[sidx], rows, sem).wait()
                pltpu.sync_copy(rows, acc.at[didx], add=True)
                return carry

            lax.fori_loop(0, SCAT_ITERS, edge_step, 0)
            plsc.subcore_barrier()

            for j in range(ROW_CHUNKS):
                sl = pl.ds(s * ROWS_PER_TILE + j * CHUNK, CHUNK)
                pltpu.sync_copy(acc.at[sl], rows)
                pltpu.sync_copy(rows, out_hbm.at[sl])

        @pl.when(c == 0)
        def _():
            run(ga_hbm, sa_hbm)

        @pl.when(c == 1)
        def _():
            run(gb_hbm, sb_hbm)

    return scatter


_scatter_h = _make_scatter(2 * HID // 2)


EPT2 = E_PAD // (NC * NS)
SCAT2_ITERS = EPT2 // CHUNK


@functools.partial(
    pl.kernel,
    out_type=(
        jax.ShapeDtypeStruct((N_PAD, HID), jnp.float32),
        jax.ShapeDtypeStruct((N_PAD, HID), jnp.float32),
    ),
    mesh=_mesh,
    scratch_types=(
        pltpu.VMEM_SHARED((N_PAD, HID), jnp.float32),
        pltpu.VMEM((CHUNK,), jnp.int32),
        pltpu.VMEM((CHUNK,), jnp.int32),
        pltpu.VMEM((CHUNK, HID), jnp.float32),
        pltpu.SemaphoreType.DMA,
    ),
)
def _scatter2(src_hbm, dst_hbm, g_hbm, sa_hbm, sb_hbm,
              acc, sidx, didx, rows, sem):
    c = lax.axis_index("c")
    s = lax.axis_index("s")

    for j in range(ROW_CHUNKS):
        sl = pl.ds(s * ROWS_PER_TILE + j * CHUNK, CHUNK)
        pltpu.sync_copy(g_hbm.at[sl], rows)
        pltpu.sync_copy(rows, acc.at[sl])
    plsc.subcore_barrier()

    def edge_step(i, carry):
        base = (c * NS + s) * EPT2 + i * CHUNK
        pltpu.sync_copy(src_hbm.at[pl.ds(base, CHUNK)], sidx)
        pltpu.sync_copy(dst_hbm.at[pl.ds(base, CHUNK)], didx)
        pltpu.async_copy(g_hbm.at[sidx], rows, sem).wait()
        pltpu.sync_copy(rows, acc.at[didx], add=True)
        return carry

    lax.fori_loop(0, SCAT2_ITERS, edge_step, 0)
    plsc.subcore_barrier()

    for j in range(ROW_CHUNKS):
        sl = pl.ds(s * ROWS_PER_TILE + j * CHUNK, CHUNK)
        pltpu.sync_copy(acc.at[sl], rows)

        @pl.when(c == 0)
        def _():
            pltpu.sync_copy(rows, sa_hbm.at[sl])

        @pl.when(c == 1)
        def _():
            pltpu.sync_copy(rows, sb_hbm.at[sl])


def _tc_layer1(dega, degb, h0, w1):
    def body(dega_ref, degb_ref, h0_ref, w1_ref, dinv_ref, ga_ref, gb_ref):
        deg = dega_ref[:] + degb_ref[:] + 1.0
        dinv = lax.rsqrt(deg)
        row = lax.broadcasted_iota(jnp.int32, (N_PAD, 1), 0)
        dinv = jnp.where(row < N, dinv, 0.0)
        dinv_ref[:] = dinv
        g = jnp.dot(h0_ref[:], w1_ref[:],
                    preferred_element_type=jnp.float32) * dinv
        ga_ref[:] = g[:, :HID]
        gb_ref[:] = g[:, HID:]

    return pl.pallas_call(
        body,
        out_shape=(
            jax.ShapeDtypeStruct((N_PAD, 1), jnp.float32),
            jax.ShapeDtypeStruct((N_PAD, HID), jnp.float32),
            jax.ShapeDtypeStruct((N_PAD, HID), jnp.float32),
        ),
    )(dega, degb, h0, w1)


def _tc_layer2(sa, sb, dinv, b1, w2):
    def body(sa_ref, sb_ref, dinv_ref, b1_ref, w2_ref, g_ref):
        s1 = jnp.concatenate([sa_ref[:], sb_ref[:]], axis=1)
        h1 = jnp.maximum(dinv_ref[:] * s1 + b1_ref[:], 0.0)
        g_ref[:] = jnp.dot(h1, w2_ref[:],
                           preferred_element_type=jnp.float32) * dinv_ref[:]

    return pl.pallas_call(
        body,
        out_shape=jax.ShapeDtypeStruct((N_PAD, HID), jnp.float32),
    )(sa, sb, dinv, b1, w2)


def _tc_out(sa, sb, g2, dinv, b2):
    def body(sa_ref, sb_ref, g2_ref, dinv_ref, b2_ref, z_ref):
        s2 = sa_ref[:] + sb_ref[:] - g2_ref[:]
        z_ref[:] = dinv_ref[:] * s2 + b2_ref[:]

    return pl.pallas_call(
        body,
        out_shape=jax.ShapeDtypeStruct((N_PAD, HID), jnp.float32),
    )(sa, sb, g2, dinv, b2)


@jax.jit
def kernel(x, edge_index, emb, W1, b1, W2, b2):
    src = edge_index[0].astype(jnp.int32)
    dst = edge_index[1].astype(jnp.int32)
    pad = jnp.full((E_PAD - E,), N, dtype=jnp.int32)
    srcp = jnp.concatenate([src, pad])
    dstp = jnp.concatenate([dst, pad])
    xp = jnp.concatenate(
        [x[:, 0].astype(jnp.int32), jnp.zeros((N_PAD - N,), jnp.int32)])

    deg0, deg1, h0 = _prep_kernel(dstp, xp, emb)
    dinv, g1a, g1b = _tc_layer1(deg0.reshape(N_PAD, 1),
                                deg1.reshape(N_PAD, 1), h0, W1)
    s1a, s1b = _scatter_h(srcp, dstp, g1a, g1b)
    g2 = _tc_layer2(s1a, s1b, dinv, b1.reshape(1, 2 * HID), W2)
    s2a, s2b = _scatter2(srcp, dstp, g2)
    z = _tc_out(s2a, s2b, g2, dinv, b2.reshape(1, HID))
    return z[:N]
